# baseline (speedup 1.0000x reference)
import jax
import jax.numpy as jnp
from jax import lax
from jax.experimental import pallas as pl
from jax.experimental.pallas import tpu as pltpu
from jax.experimental.pallas import tpu_sc as plsc

N_TRACKS = 1000000
DIM = 64
BATCH = 4096
HIST = 50

NC = 2
NS = 16
NW = NC * NS
PER_W = BATCH // NW
NBUF = 8


def _gather_body(idx_hbm, table_hbm, out_hbm, idx_v, bufs, gsem, osem):
    wid = lax.axis_index("s") * NC + lax.axis_index("c")
    base = wid * PER_W

    pltpu.sync_copy(idx_hbm.at[pl.ds(pl.multiple_of(base, PER_W), PER_W)],
                    idx_v)

    def fire(c, b):
        pltpu.async_copy(table_hbm.at[idx_v.at[c]], bufs.at[b], gsem.at[b])

    for c in range(NBUF):
        fire(c, c)

    @pl.loop(0, PER_W, step=NBUF)
    def _ring(c0):
        for b in range(NBUF):
            c = c0 + b
            pltpu.make_async_copy(
                table_hbm.at[idx_v.at[c]], bufs.at[b], gsem.at[b]
            ).wait()
            pltpu.async_copy(
                bufs.at[b], out_hbm.at[base + c], osem.at[b]
            )

            @pl.when(c + NBUF < PER_W)
            def _refill():
                pltpu.make_async_copy(
                    bufs.at[b], out_hbm.at[0], osem.at[b]
                ).wait()
                fire(c + NBUF, b)

    for b in range(NBUF):
        pltpu.make_async_copy(
            bufs.at[b], out_hbm.at[0], osem.at[b]
        ).wait()


@jax.jit
def kernel(indices_, table):
    mesh = plsc.VectorSubcoreMesh(
        core_axis_name="c", subcore_axis_name="s", num_cores=NC, num_subcores=NS
    )
    return pl.kernel(
        _gather_body,
        out_type=jax.ShapeDtypeStruct((BATCH, HIST, DIM), jnp.float32),
        mesh=mesh,
        scratch_types=[
            pltpu.VMEM((PER_W, HIST), jnp.int32),
            pltpu.VMEM((NBUF, HIST, DIM), jnp.float32),
            pltpu.SemaphoreType.DMA((NBUF,)),
            pltpu.SemaphoreType.DMA((NBUF,)),
        ],
        compiler_params=pltpu.CompilerParams(use_tc_tiling_on_sc=False),
    )(indices_, table)
